# baseline (speedup 1.0000x reference)
import jax
import jax.numpy as jnp
from jax.experimental import pallas as pl
from jax.experimental.pallas import tpu as pltpu

_N = 64
_MIN_DEPTH = 2.0
_MAX_DEPTH = 6.0
_BLKB = 32768
_NBLK = 8
_LANES = 128


def _raysample_kernel(o_ref, d_ref, pts_ref, len_ref):
    j = pl.program_id(1)
    step = (_MAX_DEPTH - _MIN_DEPTH) / (_N - 1)
    nidx = j * _NBLK + jax.lax.broadcasted_iota(jnp.int32, (1, _NBLK, 1), 1)
    z = _MIN_DEPTH + step * nidx.astype(jnp.float32)
    o = o_ref[...]
    d = d_ref[...]
    pts_ref[...] = o[:, None, :] + d[:, None, :] * z
    zl = _MIN_DEPTH + step * (
        j * _NBLK + jax.lax.broadcasted_iota(jnp.int32, (_NBLK, 1, 1), 0)
    ).astype(jnp.float32)
    len_ref[...] = jnp.broadcast_to(zl, len_ref.shape)


@jax.jit
def kernel(origins, directions):
    B = origins.shape[0]
    o_t = origins.T
    d_t = directions.T
    pts_t, len_t = pl.pallas_call(
        _raysample_kernel,
        grid=(B // _BLKB, _N // _NBLK),
        in_specs=[
            pl.BlockSpec((3, _BLKB), lambda i, j: (0, i)),
            pl.BlockSpec((3, _BLKB), lambda i, j: (0, i)),
        ],
        out_specs=[
            pl.BlockSpec((3, _NBLK, _BLKB), lambda i, j: (0, j, i)),
            pl.BlockSpec(
                (_NBLK, _BLKB // _LANES, _LANES), lambda i, j: (j, i, 0)
            ),
        ],
        out_shape=[
            jax.ShapeDtypeStruct((3, _N, B), jnp.float32),
            jax.ShapeDtypeStruct((_N, B // _LANES, _LANES), jnp.float32),
        ],
    )(o_t, d_t)
    pts = jnp.transpose(pts_t, (2, 1, 0))
    lengths = jax.lax.reshape(len_t, (B, _N, 1), dimensions=(1, 2, 0))
    return pts, lengths


# speedup vs baseline: 1.1143x; 1.1143x over previous
import jax
import jax.numpy as jnp
from jax.experimental import pallas as pl

_N = 64
_MIN_DEPTH = 2.0
_MAX_DEPTH = 6.0
_BLKB = 8192
_LANES = 128


def _points_kernel(o_ref, d_ref, pts_ref):
    step = (_MAX_DEPTH - _MIN_DEPTH) / (_N - 1)
    z = _MIN_DEPTH + step * jax.lax.broadcasted_iota(
        jnp.int32, (1, _N, 1), 1
    ).astype(jnp.float32)
    o = o_ref[...]
    d = d_ref[...]
    pts_ref[...] = o[:, None, :] + d[:, None, :] * z


def _len_kernel(len_ref):
    step = (_MAX_DEPTH - _MIN_DEPTH) / (_N - 1)
    zl = _MIN_DEPTH + step * jax.lax.broadcasted_iota(
        jnp.int32, (_N, 1, 1), 0
    ).astype(jnp.float32)
    len_ref[...] = jnp.broadcast_to(zl, len_ref.shape)


@jax.jit
def kernel(origins, directions):
    B = origins.shape[0]
    o_t = origins.T
    d_t = directions.T
    pts_t = pl.pallas_call(
        _points_kernel,
        grid=(B // _BLKB,),
        in_specs=[
            pl.BlockSpec((3, _BLKB), lambda i: (0, i)),
            pl.BlockSpec((3, _BLKB), lambda i: (0, i)),
        ],
        out_specs=pl.BlockSpec((3, _N, _BLKB), lambda i: (0, 0, i)),
        out_shape=jax.ShapeDtypeStruct((3, _N, B), jnp.float32),
    )(o_t, d_t)
    len_t = pl.pallas_call(
        _len_kernel,
        grid=(B // _BLKB,),
        out_specs=pl.BlockSpec(
            (_N, _BLKB // _LANES, _LANES), lambda i: (0, i, 0)
        ),
        out_shape=jax.ShapeDtypeStruct((_N, B // _LANES, _LANES), jnp.float32),
    )()
    pts = jnp.transpose(pts_t, (2, 1, 0))
    lengths = jax.lax.reshape(len_t, (B, _N, 1), dimensions=(1, 2, 0))
    return pts, lengths
